# Initial kernel scaffold; baseline (speedup 1.0000x reference)
import jax
import jax.numpy as jnp
from jax.experimental import pallas as pl


def kernel(x, edge_index, W, b):
    raise NotImplementedError("write your pallas kernel here")


# speedup vs baseline: 6.7890x; 6.7890x over previous
import functools

import jax
import jax.numpy as jnp
from jax import lax
from jax.experimental import pallas as pl
from jax.experimental.pallas import tpu as pltpu
from jax.experimental.pallas import tpu_sc as plsc

N_NODES = 10000
N_EDGES = 320000
D = 128

NC = 2
NS = 16
NW = NC * NS

CHUNK = 128
N_CHUNKS = N_EDGES // CHUNK
CHUNKS_PER_W = -(-N_CHUNKS // NW)

ROW_STRIDE = 624
ROW_SPAN = 640
ZROWS = 128


def _sc_gather_segment_sum(x, src, dst):
    mesh = plsc.VectorSubcoreMesh(core_axis_name="c", subcore_axis_name="s")

    @functools.partial(
        pl.kernel,
        out_type=jax.ShapeDtypeStruct((NC * N_NODES, D), jnp.float32),
        mesh=mesh,
        scratch_types=[
            pltpu.VMEM((CHUNK,), jnp.int32),
            pltpu.VMEM((CHUNK,), jnp.int32),
            pltpu.VMEM((CHUNK, D), jnp.float32),
            pltpu.VMEM((ZROWS, D), jnp.float32),
            pltpu.VMEM_SHARED((N_NODES, D), jnp.float32),
            pltpu.SemaphoreType.DMA,
        ],
    )
    def k(x_hbm, src_hbm, dst_hbm, out_hbm, src_v, dst_v, msg_v, zero_v, h_sh, sem):
        cid = lax.axis_index("c")
        sid = lax.axis_index("s")
        wid = cid * NS + sid

        @pl.loop(0, ZROWS)
        def _(r):
            @pl.loop(0, D, step=16)
            def _(f):
                zero_v[r, pl.ds(f, 16)] = jnp.zeros((16,), jnp.float32)

        @pl.loop(0, ROW_SPAN, step=ZROWS)
        def _(r0):
            pltpu.sync_copy(zero_v, h_sh.at[pl.ds(sid * ROW_STRIDE + r0, ZROWS)])

        plsc.subcore_barrier()

        @pl.loop(0, CHUNKS_PER_W)
        def _(i):
            c = wid + i * NW

            @pl.when(c < N_CHUNKS)
            def _():
                base = c * CHUNK
                pltpu.sync_copy(src_hbm.at[pl.ds(base, CHUNK)], src_v)
                pltpu.sync_copy(dst_hbm.at[pl.ds(base, CHUNK)], dst_v)
                pltpu.async_copy(x_hbm.at[src_v], msg_v, sem).wait()
                pltpu.sync_copy(msg_v, h_sh.at[dst_v], add=True)

        plsc.subcore_barrier()

        row0 = sid * ROW_STRIDE
        pltpu.sync_copy(
            h_sh.at[pl.ds(row0, ROW_SPAN)],
            out_hbm.at[pl.ds(cid * N_NODES + row0, ROW_SPAN)],
        )

    return k(x, src, dst)


def _tc_linear_relu(parts, W, b):
    BLK = 1000

    def body(p0_ref, p1_ref, w_ref, b_ref, o_ref):
        h = p0_ref[...] + p1_ref[...]
        y = lax.dot_general(
            h, w_ref[...], (((1,), (1,)), ((), ())),
            preferred_element_type=jnp.float32,
        )
        o_ref[...] = jnp.maximum(y + b_ref[...], 0.0)

    nblk = N_NODES // BLK
    return pl.pallas_call(
        body,
        grid=(nblk,),
        in_specs=[
            pl.BlockSpec((BLK, D), lambda i: (i, 0)),
            pl.BlockSpec((BLK, D), lambda i: (i + nblk, 0)),
            pl.BlockSpec((D, D), lambda i: (0, 0)),
            pl.BlockSpec((1, D), lambda i: (0, 0)),
        ],
        out_specs=pl.BlockSpec((BLK, D), lambda i: (i, 0)),
        out_shape=jax.ShapeDtypeStruct((N_NODES, D), jnp.float32),
    )(parts, parts, W, b.reshape(1, D))


def kernel(x, edge_index, W, b):
    src = edge_index[0]
    dst = edge_index[1]
    parts = _sc_gather_segment_sum(x, src, dst)
    return _tc_linear_relu(parts, W, b)


# speedup vs baseline: 12.5467x; 1.8481x over previous
import functools

import jax
import jax.numpy as jnp
from jax import lax
from jax.experimental import pallas as pl
from jax.experimental.pallas import tpu as pltpu
from jax.experimental.pallas import tpu_sc as plsc

N_NODES = 10000
N_EDGES = 320000
D = 128

NC = 2
NS = 16
NW = NC * NS

CHUNK = 128
N_CHUNKS = N_EDGES // CHUNK
N_ITER = 80

ROW_STRIDE = 624
ROW_SPAN = 640
ZROWS = 128


def _sc_gather_segment_sum(x, e3):
    mesh = plsc.VectorSubcoreMesh(core_axis_name="c", subcore_axis_name="s")

    @functools.partial(
        pl.kernel,
        out_type=jax.ShapeDtypeStruct((NC * N_NODES, D), jnp.float32),
        mesh=mesh,
        scratch_types=[
            pltpu.VMEM((2, CHUNK), jnp.int32),
            pltpu.VMEM((2, CHUNK), jnp.int32),
            pltpu.VMEM((2, CHUNK), jnp.int32),
            pltpu.VMEM((2, CHUNK), jnp.int32),
            pltpu.VMEM((CHUNK, D), jnp.float32),
            pltpu.VMEM((CHUNK, D), jnp.float32),
            pltpu.VMEM((ZROWS, D), jnp.float32),
            pltpu.VMEM_SHARED((N_NODES, D), jnp.float32),
            pltpu.SemaphoreType.DMA,
            pltpu.SemaphoreType.DMA,
            pltpu.SemaphoreType.DMA,
            pltpu.SemaphoreType.DMA,
            pltpu.SemaphoreType.DMA,
            pltpu.SemaphoreType.DMA,
            pltpu.SemaphoreType.DMA,
            pltpu.SemaphoreType.DMA,
        ],
    )
    def k(x_hbm, e_hbm, out_hbm,
          i0, i1, i2, i3, m0, m1, zero_v, h_sh,
          si0, si1, si2, si3, sg0, sg1, ss0, ss1):
        idx = [i0, i1, i2, i3]
        msg = [m0, m1]
        isem = [si0, si1, si2, si3]
        gsem = [sg0, sg1]
        ssem = [ss0, ss1]

        cid = lax.axis_index("c")
        sid = lax.axis_index("s")
        wid = cid * NS + sid

        @pl.loop(0, ZROWS)
        def _(r):
            @pl.loop(0, D, step=16)
            def _(f):
                zero_v[r, pl.ds(f, 16)] = jnp.zeros((16,), jnp.float32)

        @pl.loop(0, ROW_SPAN, step=ZROWS)
        def _(r0):
            pltpu.sync_copy(zero_v, h_sh.at[pl.ds(sid * ROW_STRIDE + r0, ZROWS)])

        plsc.subcore_barrier()

        def chunk_of(kk):
            return wid + kk * NW

        def valid(kk):
            return chunk_of(kk) < N_CHUNKS

        def start_idx(kk, b4):
            pltpu.async_copy(e_hbm.at[chunk_of(kk)], idx[b4], isem[b4])

        def wait_idx(kk, b4):
            pltpu.make_async_copy(e_hbm.at[chunk_of(kk)], idx[b4], isem[b4]).wait()

        def start_gather(b2, b4):
            pltpu.async_copy(x_hbm.at[idx[b4].at[0]], msg[b2], gsem[b2])

        def wait_gather(b2, b4):
            pltpu.make_async_copy(x_hbm.at[idx[b4].at[0]], msg[b2], gsem[b2]).wait()

        def start_scat(b2, b4):
            pltpu.async_copy(msg[b2], h_sh.at[idx[b4].at[1]], ssem[b2], add=True)

        def wait_scat(b2, b4):
            pltpu.make_async_copy(msg[b2], h_sh.at[idx[b4].at[1]], ssem[b2]).wait()

        for kk in range(3):
            start_idx(kk, kk)
        wait_idx(0, 0)
        start_gather(0, 0)

        @pl.loop(0, N_ITER // 4)
        def _(t):
            for j2 in range(4):
                kk = t * 4 + j2
                b2, b4 = j2 % 2, j2
                nb2 = 1 - b2
                pb4 = (j2 + 3) % 4
                nb4 = (j2 + 1) % 4

                @pl.when(valid(kk))
                def _():
                    wait_gather(b2, b4)
                    start_scat(b2, b4)

                if j2 == 0:
                    prev_done = (t > 0) & valid(kk - 1)
                else:
                    prev_done = valid(kk - 1)

                @pl.when(prev_done)
                def _():
                    wait_scat(nb2, pb4)

                @pl.when(valid(kk + 3))
                def _():
                    start_idx(kk + 3, pb4)

                @pl.when(valid(kk + 1))
                def _():
                    wait_idx(kk + 1, nb4)
                    start_gather(nb2, nb4)

        plsc.subcore_barrier()

        row0 = sid * ROW_STRIDE
        pltpu.sync_copy(
            h_sh.at[pl.ds(row0, ROW_SPAN)],
            out_hbm.at[pl.ds(cid * N_NODES + row0, ROW_SPAN)],
        )

    return k(x, e3)


def _tc_linear_relu(parts, W, b):
    BLK = 1000

    def body(p0_ref, p1_ref, w_ref, b_ref, o_ref):
        h = p0_ref[...] + p1_ref[...]
        y = lax.dot_general(
            h, w_ref[...], (((1,), (1,)), ((), ())),
            preferred_element_type=jnp.float32,
        )
        o_ref[...] = jnp.maximum(y + b_ref[...], 0.0)

    nblk = N_NODES // BLK
    return pl.pallas_call(
        body,
        grid=(nblk,),
        in_specs=[
            pl.BlockSpec((BLK, D), lambda i: (i, 0)),
            pl.BlockSpec((BLK, D), lambda i: (i + nblk, 0)),
            pl.BlockSpec((D, D), lambda i: (0, 0)),
            pl.BlockSpec((1, D), lambda i: (0, 0)),
        ],
        out_specs=pl.BlockSpec((BLK, D), lambda i: (i, 0)),
        out_shape=jax.ShapeDtypeStruct((N_NODES, D), jnp.float32),
    )(parts, parts, W, b.reshape(1, D))


def kernel(x, edge_index, W, b):
    e3 = edge_index.reshape(2, N_CHUNKS, CHUNK).transpose((1, 0, 2))
    parts = _sc_gather_segment_sum(x, e3)
    return _tc_linear_relu(parts, W, b)


# speedup vs baseline: 15.2548x; 1.2158x over previous
import functools

import jax
import jax.numpy as jnp
from jax import lax
from jax.experimental import pallas as pl
from jax.experimental.pallas import tpu as pltpu
from jax.experimental.pallas import tpu_sc as plsc

N_NODES = 10000
N_EDGES = 320000
D = 128

NC = 2
NS = 16
NW = NC * NS

CHUNK = 128
N_CHUNKS = N_EDGES // CHUNK
N_ITER = 84

ROW_STRIDE = 624
ROW_SPAN = 640
ZROWS = 128


def _sc_gather_segment_sum(x, e3):
    mesh = plsc.VectorSubcoreMesh(core_axis_name="c", subcore_axis_name="s")

    @functools.partial(
        pl.kernel,
        out_type=jax.ShapeDtypeStruct((NC * N_NODES, D), jnp.float32),
        mesh=mesh,
        scratch_types=(
            [pltpu.VMEM((2, CHUNK), jnp.int32)] * 6
            + [pltpu.VMEM((CHUNK, D), jnp.float32)] * 3
            + [pltpu.VMEM_SHARED((N_NODES, D), jnp.float32)]
            + [pltpu.SemaphoreType.DMA] * 12
        ),
    )
    def k(x_hbm, e_hbm, out_hbm, *refs):
        idx = list(refs[0:6])
        msg = list(refs[6:9])
        h_sh = refs[9]
        isem = list(refs[10:16])
        gsem = list(refs[16:19])
        ssem = list(refs[19:22])

        cid = lax.axis_index("c")
        sid = lax.axis_index("s")
        wid = cid * NS + sid

        @pl.loop(0, ZROWS)
        def _(r):
            @pl.loop(0, D, step=16)
            def _(f):
                msg[0][r, pl.ds(f, 16)] = jnp.zeros((16,), jnp.float32)

        @pl.loop(0, ROW_SPAN, step=ZROWS)
        def _(r0):
            pltpu.sync_copy(msg[0], h_sh.at[pl.ds(sid * ROW_STRIDE + r0, ZROWS)])

        plsc.subcore_barrier()

        def chunk_of(kk):
            return wid + kk * NW

        def valid(kk):
            return chunk_of(kk) < N_CHUNKS

        def start_idx(kk, b4):
            pltpu.async_copy(e_hbm.at[chunk_of(kk)], idx[b4], isem[b4])

        def wait_idx(kk, b4):
            pltpu.make_async_copy(e_hbm.at[chunk_of(kk)], idx[b4], isem[b4]).wait()

        def start_gather(b2, b4):
            pltpu.async_copy(x_hbm.at[idx[b4].at[0]], msg[b2], gsem[b2])

        def wait_gather(b2, b4):
            pltpu.make_async_copy(x_hbm.at[idx[b4].at[0]], msg[b2], gsem[b2]).wait()

        def start_scat(b2, b4):
            pltpu.async_copy(msg[b2], h_sh.at[idx[b4].at[1]], ssem[b2], add=True)

        def wait_scat(b2, b4):
            pltpu.make_async_copy(msg[b2], h_sh.at[idx[b4].at[1]], ssem[b2]).wait()

        for kk in range(4):
            start_idx(kk, kk)
        wait_idx(0, 0)
        start_gather(0, 0)
        wait_idx(1, 1)
        start_gather(1, 1)

        @pl.loop(0, N_ITER // 6)
        def _(t):
            for j in range(6):
                kk = t * 6 + j
                b3, b6 = j % 3, j

                @pl.when(valid(kk))
                def _():
                    wait_gather(b3, b6)
                    start_scat(b3, b6)

                if j == 0:
                    prev_done = (t > 0) & valid(kk - 1)
                else:
                    prev_done = valid(kk - 1)

                @pl.when(prev_done)
                def _():
                    wait_scat((j + 2) % 3, (j + 5) % 6)

                @pl.when(valid(kk + 4))
                def _():
                    start_idx(kk + 4, (j + 4) % 6)

                @pl.when(valid(kk + 2))
                def _():
                    wait_idx(kk + 2, (j + 2) % 6)
                    start_gather((j + 2) % 3, (j + 2) % 6)

        plsc.subcore_barrier()

        row0 = sid * ROW_STRIDE
        pltpu.sync_copy(
            h_sh.at[pl.ds(row0, ROW_SPAN)],
            out_hbm.at[pl.ds(cid * N_NODES + row0, ROW_SPAN)],
        )

    return k(x, e3)


def _tc_linear_relu(parts, W, b):
    BLK = 1000

    def body(p0_ref, p1_ref, w_ref, b_ref, o_ref):
        h = p0_ref[...] + p1_ref[...]
        y = lax.dot_general(
            h, w_ref[...], (((1,), (1,)), ((), ())),
            preferred_element_type=jnp.float32,
        )
        o_ref[...] = jnp.maximum(y + b_ref[...], 0.0)

    nblk = N_NODES // BLK
    return pl.pallas_call(
        body,
        grid=(nblk,),
        in_specs=[
            pl.BlockSpec((BLK, D), lambda i: (i, 0)),
            pl.BlockSpec((BLK, D), lambda i: (i + nblk, 0)),
            pl.BlockSpec((D, D), lambda i: (0, 0)),
            pl.BlockSpec((1, D), lambda i: (0, 0)),
        ],
        out_specs=pl.BlockSpec((BLK, D), lambda i: (i, 0)),
        out_shape=jax.ShapeDtypeStruct((N_NODES, D), jnp.float32),
    )(parts, parts, W, b.reshape(1, D))


def kernel(x, edge_index, W, b):
    e3 = edge_index.reshape(2, N_CHUNKS, CHUNK).transpose((1, 0, 2))
    parts = _sc_gather_segment_sum(x, e3)
    return _tc_linear_relu(parts, W, b)
